# baseline (device time: 31781 ns/iter reference)
import jax
import jax.numpy as jnp
from jax import lax
from jax.experimental import pallas as pl
from jax.experimental.pallas import tpu as pltpu

N_DEV = 8
N_GLOBAL = 8192
EPS = 1e-5
PC = 128
MB = 512


def _allreduce_rstd(x):
    m, n_per = x.shape
    pr, nb = m // PC, m // MB

    def body(x_ref, out_ref, rs_ref, comm_ref, send_sems, recv_sems):
        i = pl.program_id(0)
        my = lax.axis_index("i")

        @pl.when(i == 0)
        def _signal():
            bar = pltpu.get_barrier_semaphore()
            for dd in range(1, N_DEV):
                peer = (my + dd) % N_DEV
                pl.semaphore_signal(
                    bar, inc=1, device_id=(peer,),
                    device_id_type=pl.DeviceIdType.MESH,
                )

        xx = x_ref[:, :]
        rs_ref[pl.ds(i * MB, MB), :] = jnp.sum(xx * xx, axis=1, keepdims=True)

        @pl.when(i == nb - 1)
        def _comm():
            r0 = lax.broadcasted_iota(jnp.int32, (m, PC), 0)
            c0 = lax.broadcasted_iota(jnp.int32, (m, PC), 1)
            mask = jnp.bitwise_and(r0, PC - 1) == c0
            bi = lax.broadcasted_iota(jnp.int32, (pr, m), 0)
            br = lax.broadcasted_iota(jnp.int32, (pr, m), 1)
            blk = (br // PC == bi).astype(jnp.float32)
            rowsum = rs_ref[:, :]
            d = jnp.where(mask, jnp.broadcast_to(rowsum, (m, PC)), 0.0)
            comm_ref[0, :, :] = jnp.dot(
                blk, d, preferred_element_type=jnp.float32
            )

            bar = pltpu.get_barrier_semaphore()
            pl.semaphore_wait(bar, N_DEV - 1)

            rdmas = []
            for dd in range(1, N_DEV):
                peer = (my + dd) % N_DEV
                rdma = pltpu.make_async_remote_copy(
                    src_ref=comm_ref.at[0],
                    dst_ref=comm_ref.at[dd],
                    send_sem=send_sems.at[dd],
                    recv_sem=recv_sems.at[dd],
                    device_id=(peer,),
                    device_id_type=pl.DeviceIdType.MESH,
                )
                rdma.start()
                rdmas.append(rdma)

            total = comm_ref[0, :, :]
            for dd in range(1, N_DEV):
                rdmas[dd - 1].wait_recv()
                total = total + comm_ref[dd, :, :]
            for dd in range(1, N_DEV):
                rdmas[dd - 1].wait_send()

            br2 = lax.broadcasted_iota(jnp.int32, (m, pr), 0)
            bi2 = lax.broadcasted_iota(jnp.int32, (m, pr), 1)
            blk_t = (br2 // PC == bi2).astype(jnp.float32)
            t2 = jnp.dot(blk_t, total, preferred_element_type=jnp.float32)
            tot_col = jnp.sum(
                jnp.where(mask, t2, 0.0), axis=1, keepdims=True
            )
            out_ref[:, :] = lax.rsqrt(tot_col / N_GLOBAL + EPS)

    return pl.pallas_call(
        body,
        grid=(nb,),
        out_shape=jax.ShapeDtypeStruct((m, 1), jnp.float32),
        in_specs=[pl.BlockSpec((MB, n_per), lambda i: (i, 0))],
        out_specs=pl.BlockSpec((m, 1), lambda i: (0, 0)),
        scratch_shapes=[
            pltpu.VMEM((m, 1), jnp.float32),
            pltpu.VMEM((N_DEV, pr, PC), jnp.float32),
            pltpu.SemaphoreType.DMA((N_DEV,)),
            pltpu.SemaphoreType.DMA((N_DEV,)),
        ],
        compiler_params=pltpu.CompilerParams(
            collective_id=0, vmem_limit_bytes=64 * 1024 * 1024
        ),
    )(x)


def _scale(x, rstd, gamma2):
    m, n_per = x.shape
    nb = m // MB

    def body(x_ref, r_ref, g_ref, out_ref):
        out_ref[:, :] = x_ref[:, :] * r_ref[:, :] * g_ref[:, :]

    return pl.pallas_call(
        body,
        grid=(nb,),
        out_shape=jax.ShapeDtypeStruct((m, n_per), x.dtype),
        in_specs=[
            pl.BlockSpec((MB, n_per), lambda i: (i, 0)),
            pl.BlockSpec((MB, 1), lambda i: (i, 0)),
            pl.BlockSpec((1, n_per), lambda i: (0, 0)),
        ],
        out_specs=pl.BlockSpec((MB, n_per), lambda i: (i, 0)),
        compiler_params=pltpu.CompilerParams(
            vmem_limit_bytes=64 * 1024 * 1024
        ),
    )(x, rstd, gamma2)


def kernel(x, gamma):
    m, n_per = x.shape
    assert m % MB == 0 and MB % PC == 0
    rstd = _allreduce_rstd(x)
    return _scale(x, rstd, gamma.reshape(1, n_per))


# device time: 26383 ns/iter; 1.2046x vs baseline; 1.2046x over previous
import jax
import jax.numpy as jnp
from jax import lax
from jax.experimental import pallas as pl
from jax.experimental.pallas import tpu as pltpu

N_DEV = 8
N_GLOBAL = 8192
EPS = 1e-5
PC = 128
MB = 512
NB = 4096 // MB


def _allreduce_sumsq(x):
    m, n_per = x.shape
    pr = m // PC

    def body(x_hbm, out_ref, xv, rs_ref, comm_ref, in_sems, send_sems,
             recv_sems):
        my = lax.axis_index("i")

        bar = pltpu.get_barrier_semaphore()
        for dd in range(1, N_DEV):
            peer = (my + dd) % N_DEV
            pl.semaphore_signal(
                bar, inc=1, device_id=(peer,),
                device_id_type=pl.DeviceIdType.MESH,
            )

        copies = []
        for b in range(NB):
            c = pltpu.make_async_copy(
                x_hbm.at[pl.ds(b * MB, MB), :],
                xv.at[pl.ds(b * MB, MB), :],
                in_sems.at[b],
            )
            c.start()
            copies.append(c)
        for b in range(NB):
            copies[b].wait()
            xb = xv[pl.ds(b * MB, MB), :]
            rs_ref[pl.ds(b * MB, MB), :] = jnp.sum(
                xb * xb, axis=1, keepdims=True
            )

        r0 = lax.broadcasted_iota(jnp.int32, (m, PC), 0)
        c0 = lax.broadcasted_iota(jnp.int32, (m, PC), 1)
        mask = jnp.bitwise_and(r0, PC - 1) == c0
        bi = lax.broadcasted_iota(jnp.int32, (pr, m), 0)
        br = lax.broadcasted_iota(jnp.int32, (pr, m), 1)
        blk = (br // PC == bi).astype(jnp.float32)
        d = jnp.where(mask, jnp.broadcast_to(rs_ref[:, :], (m, PC)), 0.0)
        comm_ref[0, :, :] = jnp.dot(blk, d, preferred_element_type=jnp.float32)

        pl.semaphore_wait(bar, N_DEV - 1)

        rdmas = []
        for dd in range(1, N_DEV):
            peer = (my + dd) % N_DEV
            rdma = pltpu.make_async_remote_copy(
                src_ref=comm_ref.at[0],
                dst_ref=comm_ref.at[dd],
                send_sem=send_sems.at[dd],
                recv_sem=recv_sems.at[dd],
                device_id=(peer,),
                device_id_type=pl.DeviceIdType.MESH,
            )
            rdma.start()
            rdmas.append(rdma)

        total = comm_ref[0, :, :]
        for dd in range(1, N_DEV):
            rdmas[dd - 1].wait_recv()
            total = total + comm_ref[dd, :, :]
        for dd in range(1, N_DEV):
            rdmas[dd - 1].wait_send()
        out_ref[:, :] = total

    return pl.pallas_call(
        body,
        out_shape=jax.ShapeDtypeStruct((pr, PC), jnp.float32),
        in_specs=[pl.BlockSpec(memory_space=pl.ANY)],
        out_specs=pl.BlockSpec(memory_space=pltpu.VMEM),
        scratch_shapes=[
            pltpu.VMEM((m, n_per), jnp.float32),
            pltpu.VMEM((m, 1), jnp.float32),
            pltpu.VMEM((N_DEV, pr, PC), jnp.float32),
            pltpu.SemaphoreType.DMA((NB,)),
            pltpu.SemaphoreType.DMA((N_DEV,)),
            pltpu.SemaphoreType.DMA((N_DEV,)),
        ],
        compiler_params=pltpu.CompilerParams(
            collective_id=0, vmem_limit_bytes=100 * 1024 * 1024
        ),
    )(x)


def _scale(x, total, gamma2):
    m, n_per = x.shape
    pr = m // PC

    def body(x_hbm, t_ref, g_ref, out_hbm, xv, ov, in_sems, out_sems):
        copies = []
        for b in range(NB):
            c = pltpu.make_async_copy(
                x_hbm.at[pl.ds(b * MB, MB), :],
                xv.at[pl.ds(b * MB, MB), :],
                in_sems.at[b],
            )
            c.start()
            copies.append(c)

        r0 = lax.broadcasted_iota(jnp.int32, (m, PC), 0)
        c0 = lax.broadcasted_iota(jnp.int32, (m, PC), 1)
        mask = jnp.bitwise_and(r0, PC - 1) == c0
        br2 = lax.broadcasted_iota(jnp.int32, (m, pr), 0)
        bi2 = lax.broadcasted_iota(jnp.int32, (m, pr), 1)
        blk_t = (br2 // PC == bi2).astype(jnp.float32)
        t2 = jnp.dot(blk_t, t_ref[:, :], preferred_element_type=jnp.float32)
        tot_col = jnp.sum(jnp.where(mask, t2, 0.0), axis=1, keepdims=True)
        rstd = lax.rsqrt(tot_col / N_GLOBAL + EPS)
        g = g_ref[:, :]

        out_copies = []
        for b in range(NB):
            copies[b].wait()
            sl = pl.ds(b * MB, MB)
            ov[sl, :] = xv[sl, :] * rstd[b * MB:(b + 1) * MB, :] * g
            oc = pltpu.make_async_copy(
                ov.at[sl, :], out_hbm.at[sl, :], out_sems.at[b]
            )
            oc.start()
            out_copies.append(oc)
        for oc in out_copies:
            oc.wait()

    return pl.pallas_call(
        body,
        out_shape=jax.ShapeDtypeStruct((m, n_per), x.dtype),
        in_specs=[
            pl.BlockSpec(memory_space=pl.ANY),
            pl.BlockSpec(memory_space=pltpu.VMEM),
            pl.BlockSpec(memory_space=pltpu.VMEM),
        ],
        out_specs=pl.BlockSpec(memory_space=pl.ANY),
        scratch_shapes=[
            pltpu.VMEM((m, n_per), jnp.float32),
            pltpu.VMEM((m, n_per), jnp.float32),
            pltpu.SemaphoreType.DMA((NB,)),
            pltpu.SemaphoreType.DMA((NB,)),
        ],
        compiler_params=pltpu.CompilerParams(
            vmem_limit_bytes=100 * 1024 * 1024
        ),
    )(x, total, gamma2)


def kernel(x, gamma):
    m, n_per = x.shape
    assert m % MB == 0 and MB % PC == 0
    total = _allreduce_sumsq(x)
    return _scale(x, total, gamma.reshape(1, n_per))
